# baseline (speedup 1.0000x reference)
import functools

import jax
import jax.numpy as jnp
from jax import lax
from jax.experimental import pallas as pl
from jax.experimental.pallas import tpu as pltpu
from jax.experimental.pallas import tpu_sc as plsc

_NUM_CORES = 2
_NUM_SUBCORES = 16
_NUM_WORKERS = _NUM_CORES * _NUM_SUBCORES
_CHUNK = 128


@functools.lru_cache(maxsize=None)
def _make_kernel(n_chunks: int, d: int):
    mesh = plsc.VectorSubcoreMesh(core_axis_name="c", subcore_axis_name="s")

    @functools.partial(
        pl.kernel,
        mesh=mesh,
        out_type=jax.ShapeDtypeStruct((_NUM_WORKERS, n_chunks, _CHUNK, d),
                                      jnp.float32),
        scratch_types=[
            pltpu.VMEM((n_chunks, _CHUNK), jnp.int32),
            pltpu.VMEM((n_chunks, _CHUNK, d), jnp.float32),
            pltpu.SemaphoreType.DMA,
        ],
        compiler_params=pltpu.CompilerParams(use_tc_tiling_on_sc=False),
    )
    def k(table_hbm, idx_hbm, out_hbm, idx_v, rows_v, sem):
        wid = lax.axis_index("s") * _NUM_CORES + lax.axis_index("c")
        pltpu.sync_copy(idx_hbm.at[wid], idx_v)
        copies = [
            pltpu.async_copy(table_hbm.at[idx_v.at[j]], rows_v.at[j], sem)
            for j in range(n_chunks)
        ]
        for c in copies:
            c.wait()
        pltpu.sync_copy(rows_v, out_hbm.at[wid])

    return k


def kernel(labels, embedding_table):
    (b,) = labels.shape
    _, d = embedding_table.shape
    n_chunks = b // (_NUM_WORKERS * _CHUNK)
    idx = labels.astype(jnp.int32).reshape(_NUM_WORKERS, n_chunks, _CHUNK)
    out = _make_kernel(n_chunks, d)(embedding_table, idx)
    return out.reshape(b, d)


# speedup vs baseline: 1.4060x; 1.4060x over previous
import functools

import jax
import jax.numpy as jnp
from jax import lax
from jax.experimental import pallas as pl
from jax.experimental.pallas import tpu as pltpu
from jax.experimental.pallas import tpu_sc as plsc

_NUM_CORES = 2
_NUM_SUBCORES = 16
_NUM_WORKERS = _NUM_CORES * _NUM_SUBCORES
_WAVE = 16


@functools.lru_cache(maxsize=None)
def _make_kernel(b: int, d: int):
    per_tile = b // _NUM_WORKERS
    n_waves = per_tile // _WAVE
    mesh = plsc.VectorSubcoreMesh(core_axis_name="c", subcore_axis_name="s")

    @functools.partial(
        pl.kernel,
        mesh=mesh,
        out_type=jax.ShapeDtypeStruct((b * d,), jnp.float32),
        scratch_types=[
            pltpu.VMEM((per_tile,), jnp.int32),
            pltpu.VMEM((_WAVE, 8, d), jnp.float32),
            pltpu.VMEM((per_tile * d,), jnp.float32),
            pltpu.SemaphoreType.DMA,
        ],
        compiler_params=pltpu.CompilerParams(needs_layout_passes=False),
    )
    def k(table_hbm, idx_hbm, out_hbm, idx_v, tile_v, stage_v, sem):
        wid = lax.axis_index("s") * _NUM_CORES + lax.axis_index("c")
        base = wid * per_tile
        pltpu.sync_copy(idx_hbm.at[pl.ds(base, per_tile)], idx_v)
        lanes = lax.iota(jnp.int32, 16)

        def wave_body(w, carry):
            wave_vec = idx_v[pl.ds(w * _WAVE, _WAVE)]
            t_vec = lax.bitwise_and(wave_vec, jnp.int32(-8))
            copies = [
                pltpu.async_copy(
                    table_hbm.at[pl.ds(pl.multiple_of(t_vec[u], 8), 8)],
                    tile_v.at[u], sem)
                for u in range(_WAVE)
            ]
            for c in copies:
                c.wait()
            s_vec = lax.bitwise_and(wave_vec, jnp.int32(7))
            addr0 = (w * _WAVE + lanes) * d
            for col in range(d):
                col_vec = jnp.full((16,), col, jnp.int32)
                val = plsc.load_gather(tile_v, [lanes, s_vec, col_vec])
                plsc.store_scatter(stage_v, [addr0 + col], val)
            return carry

        lax.fori_loop(0, n_waves, wave_body, 0)
        pltpu.sync_copy(stage_v, out_hbm.at[pl.ds(base * d, per_tile * d)])

    return k


def kernel(labels, embedding_table):
    (b,) = labels.shape
    _, d = embedding_table.shape
    out = _make_kernel(b, d)(embedding_table, labels.astype(jnp.int32))
    return out.reshape(b, d)


# speedup vs baseline: 2.3151x; 1.6465x over previous
import functools

import jax
import jax.numpy as jnp
from jax import lax
from jax.experimental import pallas as pl
from jax.experimental.pallas import tpu as pltpu
from jax.experimental.pallas import tpu_sc as plsc

_NUM_CORES = 2
_NUM_SUBCORES = 16
_NUM_WORKERS = _NUM_CORES * _NUM_SUBCORES
_GRP = 16
_SUB = 4


@functools.lru_cache(maxsize=None)
def _make_kernel(b: int, d: int):
    per_tile = b // _NUM_WORKERS
    n_groups = per_tile // _GRP
    stage_sz = _GRP * d
    mesh = plsc.VectorSubcoreMesh(core_axis_name="c", subcore_axis_name="s")

    @functools.partial(
        pl.kernel,
        mesh=mesh,
        out_type=jax.ShapeDtypeStruct((b * d,), jnp.float32),
        scratch_types=[
            pltpu.VMEM((per_tile,), jnp.int32),
            pltpu.VMEM((2, _SUB, d, 128), jnp.float32),
            pltpu.VMEM((2 * stage_sz,), jnp.float32),
            pltpu.SemaphoreType.DMA,
            pltpu.SemaphoreType.DMA,
            pltpu.SemaphoreType.DMA,
        ],
        compiler_params=pltpu.CompilerParams(needs_layout_passes=False,
                                             disable_bounds_checks=True),
    )
    def k(table_hbm, idx_hbm, out_hbm, idx_v, col_v, stage_v, sem_a, sem_b,
          sem_w):
        wid = lax.axis_index("s") * _NUM_CORES + lax.axis_index("c")
        base = wid * per_tile
        pltpu.sync_copy(idx_hbm.at[pl.ds(base, per_tile)], idx_v)
        lanes = lax.iota(jnp.int32, 16)
        sems = [sem_a, sem_b]

        def fire(wave_vec, s, ping):
            copies = []
            for u in range(_SUB):
                idx = wave_vec[s * _SUB + u]
                off = lax.shift_left(lax.shift_right_logical(idx, 7), 7)
                copies.append(pltpu.async_copy(
                    table_hbm.at[:, pl.ds(pl.multiple_of(off, 128), 128)],
                    col_v.at[ping, u], sems[ping]))
            return copies

        def extract(wave_vec, s, ping, stage_off):
            for u in range(_SUB):
                idx = wave_vec[s * _SUB + u]
                l_vec = jnp.full((16,), lax.bitwise_and(idx, jnp.int32(127)),
                                 jnp.int32)
                p_vec = jnp.full((16,), ping, jnp.int32)
                u_vec = jnp.full((16,), u, jnp.int32)
                row0 = stage_off + (s * _SUB + u) * d
                for g in range(d // 16):
                    d_vec = lanes + g * 16
                    val = plsc.load_gather(col_v, [p_vec, u_vec, d_vec, l_vec])
                    plsc.store_scatter(
                        stage_v, [jnp.full((16,), row0 + g * 16, jnp.int32)
                                  + lanes], val)

        def wb_drain(stage_off):
            pltpu.make_async_copy(
                out_hbm.at[pl.ds(0, stage_sz)],
                stage_v.at[pl.ds(stage_off, stage_sz)], sem_w).wait()

        def body(w, carry):
            wave_vec = idx_v[pl.ds(w * _GRP, _GRP)]
            stage_off = lax.rem(w, 2) * stage_sz

            @pl.when(w >= 2)
            def _():
                wb_drain(stage_off)

            nsub = _GRP // _SUB
            prev = fire(wave_vec, 0, 0)
            prev_ping = 0
            for s in range(nsub):
                nxt = (fire(wave_vec, s + 1, 1 - prev_ping)
                       if s + 1 < nsub else None)
                for c in prev:
                    c.wait()
                extract(wave_vec, s, prev_ping, stage_off)
                if nxt is not None:
                    prev = nxt
                    prev_ping = 1 - prev_ping
            pltpu.async_copy(
                stage_v.at[pl.ds(stage_off, stage_sz)],
                out_hbm.at[pl.ds((base + w * _GRP) * d, stage_sz)], sem_w)
            return carry

        lax.fori_loop(0, n_groups, body, 0)
        wb_drain(0)
        wb_drain(stage_sz)

    return k


def kernel(labels, embedding_table):
    (b,) = labels.shape
    _, d = embedding_table.shape
    out = _make_kernel(b, d)(embedding_table.T, labels.astype(jnp.int32))
    return out.reshape(b, d)


# speedup vs baseline: 3.4004x; 1.4688x over previous
import functools

import jax
import jax.numpy as jnp
from jax import lax
from jax.experimental import pallas as pl
from jax.experimental.pallas import tpu as pltpu
from jax.experimental.pallas import tpu_sc as plsc

_NUM_CORES = 2
_NUM_SUBCORES = 16
_NUM_WORKERS = _NUM_CORES * _NUM_SUBCORES
_LANE = 128
_STAGE = 128


@functools.lru_cache(maxsize=None)
def _make_kernel(b: int, d: int, v: int):
    n_cols = -(-v // _LANE)
    cols_per = -(-n_cols // _NUM_WORKERS)
    n_blocks = -(-cols_per // 16)
    out_rows = b + _NUM_WORKERS
    mesh = plsc.VectorSubcoreMesh(core_axis_name="c", subcore_axis_name="s")

    @functools.partial(
        pl.kernel,
        mesh=mesh,
        out_type=jax.ShapeDtypeStruct((out_rows, _LANE), jnp.float32),
        scratch_types=[
            pltpu.VMEM((b + 16,), jnp.int32),
            pltpu.VMEM((b + 16,), jnp.int32),
            pltpu.VMEM((b + 256,), jnp.int32),
            pltpu.VMEM((1, 32), jnp.int32),
            pltpu.VMEM((2 * d, 2 * _LANE), jnp.float32),
            pltpu.VMEM((_STAGE, _LANE), jnp.float32),
            pltpu.VMEM((1, _STAGE), jnp.int32),
            pltpu.SemaphoreType.DMA,
            pltpu.SemaphoreType.DMA,
            pltpu.SemaphoreType.DMA,
        ],
        compiler_params=pltpu.CompilerParams(needs_layout_passes=False,
                                             disable_bounds_checks=True),
    )
    def k(table_hbm, idx_hbm, out_hbm, lab_v, my_pos, bk_pos, starts_v,
          col_v, stage_v, spos_v, sem_a, sem_b, sem_s):
        wid = lax.axis_index("s") * _NUM_CORES + lax.axis_index("c")
        lanes = lax.iota(jnp.int32, 16)
        zeros16 = jnp.zeros((16,), jnp.int32)
        lo_col = wid * cols_per
        hi_col = jnp.minimum(lo_col + cols_per, n_cols)
        n_mine = hi_col - lo_col
        lo_lab = jnp.full((16,), lo_col * _LANE, jnp.int32)
        hi_lab = jnp.full((16,), hi_col * _LANE, jnp.int32)
        trash = jnp.full((16,), b + wid, jnp.int32)
        sent_pos = jnp.full((16,), b, jnp.int32)
        lane0 = lanes == 0
        sems = [sem_a, sem_b]

        pltpu.sync_copy(idx_hbm, lab_v.at[pl.ds(0, b)])
        lab_v[pl.ds(b, 16)] = jnp.full((16,), jnp.int32(0x7f000000), jnp.int32)

        def pf(vi, cnt):
            lv = lab_v[pl.ds(vi * 16, 16)]
            m = jnp.logical_and(lv >= lo_lab, lv < hi_lab)
            plsc.store_compressed(my_pos.at[pl.ds(cnt, 16)],
                                  jnp.full((16,), vi * 16, jnp.int32) + lanes,
                                  mask=m)
            return cnt + plsc.all_reduce_population_count(m)[0]

        cnt = lax.fori_loop(0, b // 16, pf, jnp.int32(0))
        my_pos[pl.ds(cnt, 16)] = sent_pos
        n_vregs = lax.div(cnt + 15, jnp.int32(16))

        def bucket(q, bcnt):
            blo = jnp.full((16,), (lo_col + q * 16) * _LANE, jnp.int32)
            bhi = jnp.full((16,), (lo_col + q * 16 + 16) * _LANE, jnp.int32)
            plsc.store_scatter(starts_v,
                               [zeros16, jnp.full((16,), q, jnp.int32)],
                               jnp.full((16,), bcnt, jnp.int32), mask=lane0)

            def bb(vi, bc):
                pv = my_pos[pl.ds(vi * 16, 16)]
                lv = plsc.load_gather(lab_v, [pv])
                m = jnp.logical_and(lv >= blo, lv < bhi)
                plsc.store_compressed(bk_pos.at[pl.ds(bc, 16)], pv, mask=m)
                return bc + plsc.all_reduce_population_count(m)[0]

            bcnt = lax.fori_loop(0, n_vregs, bb, bcnt)
            bk_pos[pl.ds(bcnt, 16)] = sent_pos
            return lax.bitwise_and(bcnt + 15, jnp.int32(~15))

        bcnt_end = lax.fori_loop(0, n_blocks, bucket, jnp.int32(0))
        plsc.store_scatter(starts_v,
                           [zeros16, jnp.full((16,), n_blocks, jnp.int32)],
                           jnp.full((16,), bcnt_end, jnp.int32), mask=lane0)

        def reset_spos():
            for g in range(_STAGE // 16):
                spos_v[0, pl.ds(g * 16, 16)] = trash

        reset_spos()

        def flush():
            pltpu.async_copy(stage_v, out_hbm.at[spos_v.at[0]], sem_s).wait()
            reset_spos()

        def fire(c2, ping):
            off = (lo_col + c2) * _LANE
            pltpu.async_copy(
                table_hbm.at[:, pl.ds(pl.multiple_of(off, _LANE), 2 * _LANE)],
                col_v.at[pl.ds(ping * d, d)], sems[ping])

        def drain(ping):
            pltpu.make_async_copy(
                table_hbm.at[:, pl.ds(0, 2 * _LANE)],
                col_v.at[pl.ds(ping * d, d)], sems[ping]).wait()

        def extract_hits(m0, vi, ping, base_vec, scnt0):
            def cond(carry):
                m, _ = carry
                return plsc.all_reduce_population_count(m)[0] > 0

            def body(carry):
                m, scnt = carry
                ln = plsc.all_reduce_ffs(m)
                idx16 = jnp.full((16,), vi * 16, jnp.int32) + ln
                pos_s = plsc.load_gather(bk_pos, [idx16])
                lab_s = plsc.load_gather(lab_v, [pos_s])
                col_par = lax.shift_right_logical(lab_s, 7) - base_vec
                l_vec = (lax.bitwise_and(lab_s, jnp.int32(_LANE - 1))
                         + col_par * _LANE)
                row_s = jnp.full((16,), scnt, jnp.int32)
                for g in range(d // 16):
                    d_vec = jnp.full((16,), ping * d + g * 16,
                                     jnp.int32) + lanes
                    val = plsc.load_gather(col_v, [d_vec, l_vec])
                    plsc.store_scatter(stage_v, [row_s, lanes + g * 16], val)
                plsc.store_scatter(spos_v, [zeros16, row_s], pos_s,
                                   mask=lane0)
                m = jnp.logical_and(m, lanes != ln)
                return m, scnt + 1

            _, scnt = lax.while_loop(cond, body, (m0, scnt0))
            return scnt

        n_pairs = lax.div(n_mine + 1, jnp.int32(2))
        fire(jnp.int32(0), 0)

        def process(kk, ping, scnt):
            drain(ping)
            base = lo_col + kk * 2
            base_vec = jnp.full((16,), base, jnp.int32)
            q_vec = lax.div(jnp.full((16,), kk * 2, jnp.int32), jnp.int32(16))
            s0 = plsc.load_gather(starts_v, [zeros16, q_vec])[0]
            s1 = plsc.load_gather(starts_v, [zeros16, q_vec + 1])[0]

            def vreg_body(vi, scnt_i):
                full = scnt_i >= _STAGE - 16

                @pl.when(full)
                def _():
                    flush()

                scnt_i = jnp.where(full, jnp.int32(0), scnt_i)
                pv = bk_pos[pl.ds(vi * 16, 16)]
                lv = plsc.load_gather(lab_v, [pv])
                cc = lax.shift_right_logical(lv, 7) - base_vec
                m = jnp.logical_and(cc >= 0, cc < 2)
                pc = plsc.all_reduce_population_count(m)[0]
                return lax.cond(
                    pc > 0,
                    lambda: extract_hits(m, vi, ping, base_vec, scnt_i),
                    lambda: scnt_i)

            return lax.fori_loop(lax.div(s0, jnp.int32(16)),
                                 lax.div(s1, jnp.int32(16)),
                                 vreg_body, scnt)

        def pair_loop(kk2, scnt):
            for ping in (0, 1):
                kk = kk2 * 2 + ping

                @pl.when(kk + 1 < n_pairs)
                def _():
                    fire((kk + 1) * 2, 1 - ping)

                scnt = lax.cond(kk < n_pairs,
                                lambda kk=kk, ping=ping, s=scnt:
                                    process(kk, ping, s),
                                lambda s=scnt: s)
            return scnt

        lax.fori_loop(0, lax.div(n_pairs + 1, jnp.int32(2)), pair_loop,
                      jnp.int32(0))
        pltpu.async_copy(stage_v, out_hbm.at[spos_v.at[0]], sem_s).wait()

    return k


def kernel(labels, embedding_table):
    (b,) = labels.shape
    v, d = embedding_table.shape
    out = _make_kernel(b, d, v)(embedding_table.T, labels.astype(jnp.int32))
    return out[:b, :d]


# speedup vs baseline: 3.6986x; 1.0877x over previous
import functools

import jax
import jax.numpy as jnp
from jax import lax
from jax.experimental import pallas as pl
from jax.experimental.pallas import tpu as pltpu
from jax.experimental.pallas import tpu_sc as plsc

_NUM_CORES = 2
_NUM_SUBCORES = 16
_NUM_WORKERS = _NUM_CORES * _NUM_SUBCORES
_LANE = 128
_STAGE = 96
_QUAD = 4


@functools.lru_cache(maxsize=None)
def _make_kernel(b: int, d: int, v: int):
    n_cols = -(-v // _LANE)
    n_cols_alloc = n_cols
    cols_per = -(-n_cols // _NUM_WORKERS)
    n_blocks = -(-cols_per // 16)
    out_rows = b + _NUM_WORKERS
    mesh = plsc.VectorSubcoreMesh(core_axis_name="c", subcore_axis_name="s")

    @functools.partial(
        pl.kernel,
        mesh=mesh,
        out_type=jax.ShapeDtypeStruct((out_rows, _LANE), jnp.float32),
        scratch_types=[
            pltpu.VMEM((b + 16,), jnp.int32),
            pltpu.VMEM((b + 16,), jnp.int32),
            pltpu.VMEM((b + 256,), jnp.int32),
            pltpu.VMEM((1, 32), jnp.int32),
            pltpu.VMEM((2 * d, _QUAD * _LANE), jnp.float32),
            pltpu.VMEM((_STAGE, _LANE), jnp.float32),
            pltpu.VMEM((1, _STAGE), jnp.int32),
            pltpu.SemaphoreType.DMA,
            pltpu.SemaphoreType.DMA,
            pltpu.SemaphoreType.DMA,
        ],
        compiler_params=pltpu.CompilerParams(needs_layout_passes=False,
                                             disable_bounds_checks=True),
    )
    def k(table_hbm, idx_hbm, out_hbm, lab_v, my_pos, bk_pos, starts_v,
          col_v, stage_v, spos_v, sem_a, sem_b, sem_s):
        wid = lax.axis_index("s") * _NUM_CORES + lax.axis_index("c")
        lanes = lax.iota(jnp.int32, 16)
        zeros16 = jnp.zeros((16,), jnp.int32)
        lo_col = wid * cols_per
        hi_col = jnp.minimum(lo_col + cols_per, n_cols)
        n_mine = hi_col - lo_col
        lo_lab = jnp.full((16,), lo_col * _LANE, jnp.int32)
        hi_lab = jnp.full((16,), hi_col * _LANE, jnp.int32)
        trash = jnp.full((16,), b + wid, jnp.int32)
        sent_pos = jnp.full((16,), b, jnp.int32)
        lane0 = lanes == 0
        sems = [sem_a, sem_b]

        pltpu.sync_copy(idx_hbm, lab_v.at[pl.ds(0, b)])
        lab_v[pl.ds(b, 16)] = jnp.full((16,), jnp.int32(0x7f000000), jnp.int32)

        def pf(vi, cnt):
            lv = lab_v[pl.ds(vi * 16, 16)]
            m = jnp.logical_and(lv >= lo_lab, lv < hi_lab)
            plsc.store_compressed(my_pos.at[pl.ds(cnt, 16)],
                                  jnp.full((16,), vi * 16, jnp.int32) + lanes,
                                  mask=m)
            return cnt + plsc.all_reduce_population_count(m)[0]

        cnt = lax.fori_loop(0, b // 16, pf, jnp.int32(0))
        my_pos[pl.ds(cnt, 16)] = sent_pos
        n_vregs = lax.div(cnt + 15, jnp.int32(16))

        def bucket(q, bcnt):
            blo = jnp.full((16,), (lo_col + q * 16) * _LANE, jnp.int32)
            bhi = jnp.full((16,), (lo_col + q * 16 + 16) * _LANE, jnp.int32)
            plsc.store_scatter(starts_v,
                               [zeros16, jnp.full((16,), q, jnp.int32)],
                               jnp.full((16,), bcnt, jnp.int32), mask=lane0)

            def bb(vi, bc):
                pv = my_pos[pl.ds(vi * 16, 16)]
                lv = plsc.load_gather(lab_v, [pv])
                m = jnp.logical_and(lv >= blo, lv < bhi)
                plsc.store_compressed(bk_pos.at[pl.ds(bc, 16)], pv, mask=m)
                return bc + plsc.all_reduce_population_count(m)[0]

            bcnt = lax.fori_loop(0, n_vregs, bb, bcnt)
            bk_pos[pl.ds(bcnt, 16)] = sent_pos
            return lax.bitwise_and(bcnt + 15, jnp.int32(~15))

        bcnt_end = lax.fori_loop(0, n_blocks, bucket, jnp.int32(0))
        plsc.store_scatter(starts_v,
                           [zeros16, jnp.full((16,), n_blocks, jnp.int32)],
                           jnp.full((16,), bcnt_end, jnp.int32), mask=lane0)

        def reset_spos():
            for g in range(_STAGE // 16):
                spos_v[0, pl.ds(g * 16, 16)] = trash

        reset_spos()

        def flush():
            pltpu.async_copy(stage_v, out_hbm.at[spos_v.at[0]], sem_s).wait()
            reset_spos()

        max_start = n_cols_alloc - _QUAD

        def fire(c4, ping):
            off_col = jnp.minimum(lo_col + c4, max_start)
            off = off_col * _LANE
            pltpu.async_copy(
                table_hbm.at[:, pl.ds(pl.multiple_of(off, _LANE),
                                      _QUAD * _LANE)],
                col_v.at[pl.ds(ping * d, d)], sems[ping])

        def drain(ping):
            pltpu.make_async_copy(
                table_hbm.at[:, pl.ds(0, _QUAD * _LANE)],
                col_v.at[pl.ds(ping * d, d)], sems[ping]).wait()

        def extract_hits(m0, vi, ping, off_vec, scnt0):
            def cond(carry):
                m, _ = carry
                return plsc.all_reduce_population_count(m)[0] > 0

            def body(carry):
                m, scnt = carry
                ln = plsc.all_reduce_ffs(m)
                idx16 = jnp.full((16,), vi * 16, jnp.int32) + ln
                pos_s = plsc.load_gather(bk_pos, [idx16])
                lab_s = plsc.load_gather(lab_v, [pos_s])
                col_par = lax.shift_right_logical(lab_s, 7) - off_vec
                l_vec = (lax.bitwise_and(lab_s, jnp.int32(_LANE - 1))
                         + col_par * _LANE)
                row_s = jnp.full((16,), scnt, jnp.int32)
                for g in range(d // 16):
                    d_vec = jnp.full((16,), ping * d + g * 16,
                                     jnp.int32) + lanes
                    val = plsc.load_gather(col_v, [d_vec, l_vec])
                    plsc.store_scatter(stage_v, [row_s, lanes + g * 16], val)
                plsc.store_scatter(spos_v, [zeros16, row_s], pos_s,
                                   mask=lane0)
                m = jnp.logical_and(m, lanes != ln)
                return m, scnt + 1

            _, scnt = lax.while_loop(cond, body, (m0, scnt0))
            return scnt

        n_pairs = lax.div(n_mine + _QUAD - 1, jnp.int32(_QUAD))
        fire(jnp.int32(0), 0)

        def process(kk, ping, scnt):
            drain(ping)
            base = lo_col + kk * _QUAD
            base_vec = jnp.full((16,), base, jnp.int32)
            off_vec = jnp.minimum(base_vec, jnp.int32(max_start))
            q_vec = lax.div(jnp.full((16,), kk * _QUAD, jnp.int32),
                            jnp.int32(16))
            s0 = plsc.load_gather(starts_v, [zeros16, q_vec])[0]
            s1 = plsc.load_gather(starts_v, [zeros16, q_vec + 1])[0]

            def vreg_body(vi, scnt_i):
                full = scnt_i >= _STAGE - 16

                @pl.when(full)
                def _():
                    flush()

                scnt_i = jnp.where(full, jnp.int32(0), scnt_i)
                pv = bk_pos[pl.ds(vi * 16, 16)]
                lv = plsc.load_gather(lab_v, [pv])
                cc = lax.shift_right_logical(lv, 7) - base_vec
                m = jnp.logical_and(cc >= 0, cc < _QUAD)
                pc = plsc.all_reduce_population_count(m)[0]
                return lax.cond(
                    pc > 0,
                    lambda: extract_hits(m, vi, ping, off_vec, scnt_i),
                    lambda: scnt_i)

            return lax.fori_loop(lax.div(s0, jnp.int32(16)),
                                 lax.div(s1, jnp.int32(16)),
                                 vreg_body, scnt)

        def pair_loop(kk2, scnt):
            for ping in (0, 1):
                kk = kk2 * 2 + ping

                @pl.when(kk + 1 < n_pairs)
                def _():
                    fire((kk + 1) * _QUAD, 1 - ping)

                scnt = lax.cond(kk < n_pairs,
                                lambda kk=kk, ping=ping, s=scnt:
                                    process(kk, ping, s),
                                lambda s=scnt: s)
            return scnt

        lax.fori_loop(0, lax.div(n_pairs + 1, jnp.int32(2)), pair_loop,
                      jnp.int32(0))
        pltpu.async_copy(stage_v, out_hbm.at[spos_v.at[0]], sem_s).wait()

    return k


def kernel(labels, embedding_table):
    (b,) = labels.shape
    v, d = embedding_table.shape
    out = _make_kernel(b, d, v)(embedding_table.T, labels.astype(jnp.int32))
    return out[:b, :d]
